# baseline (speedup 1.0000x reference)
import jax
import jax.numpy as jnp
from jax.experimental import pallas as pl

N_VARS = 64
EMBED_DIM = 64


_ROWS_PER_BLOCK = 256


def _bcast_kernel(emb_ref, out_ref):
    out_ref[...] = jnp.broadcast_to(emb_ref[...], out_ref.shape)


def kernel(x, channel_emb):
    B, n_patches, _ = x.shape
    flat = N_VARS * EMBED_DIM
    n_rows = B * n_patches
    emb_flat = channel_emb.reshape(1, flat)
    out2d = pl.pallas_call(
        _bcast_kernel,
        grid=(n_rows // _ROWS_PER_BLOCK,),
        in_specs=[pl.BlockSpec((1, flat), lambda i: (0, 0))],
        out_specs=pl.BlockSpec((_ROWS_PER_BLOCK, flat), lambda i: (i, 0)),
        out_shape=jax.ShapeDtypeStruct((n_rows, flat), channel_emb.dtype),
    )(emb_flat)
    return out2d.reshape(B, n_patches, N_VARS, EMBED_DIM)


# speedup vs baseline: 1.0077x; 1.0077x over previous
import jax
import jax.numpy as jnp
from jax.experimental import pallas as pl
from jax.experimental.pallas import tpu as pltpu

N_VARS = 64
EMBED_DIM = 64
_ROWS_PER_COPY = 512


def _bcast_kernel(emb_ref, out_ref, scratch_ref, sem):
    scratch_ref[...] = jnp.broadcast_to(emb_ref[...], scratch_ref.shape)
    n_rows = out_ref.shape[0]
    nchunks = n_rows // _ROWS_PER_COPY
    for i in range(nchunks):
        pltpu.make_async_copy(
            scratch_ref,
            out_ref.at[pl.ds(i * _ROWS_PER_COPY, _ROWS_PER_COPY), :],
            sem,
        ).start()
    for i in range(nchunks):
        pltpu.make_async_copy(
            scratch_ref,
            out_ref.at[pl.ds(i * _ROWS_PER_COPY, _ROWS_PER_COPY), :],
            sem,
        ).wait()


def kernel(x, channel_emb):
    B, n_patches, _ = x.shape
    flat = N_VARS * EMBED_DIM
    n_rows = B * n_patches
    emb_flat = channel_emb.reshape(1, flat)
    out2d = pl.pallas_call(
        _bcast_kernel,
        in_specs=[pl.BlockSpec(memory_space=pltpu.VMEM)],
        out_specs=pl.BlockSpec(memory_space=pl.ANY),
        out_shape=jax.ShapeDtypeStruct((n_rows, flat), channel_emb.dtype),
        scratch_shapes=[
            pltpu.VMEM((_ROWS_PER_COPY, flat), channel_emb.dtype),
            pltpu.SemaphoreType.DMA,
        ],
    )(emb_flat)
    return out2d.reshape(B, n_patches, N_VARS, EMBED_DIM)


# speedup vs baseline: 7.2641x; 7.2082x over previous
import jax
import jax.numpy as jnp
from jax.experimental import pallas as pl

N_VARS = 64
EMBED_DIM = 64


def _bcast_kernel(col_ref, out_ref):
    out_ref[...] = jnp.broadcast_to(col_ref[...], out_ref.shape)


def kernel(x, channel_emb):
    B, n_patches, _ = x.shape
    flat = N_VARS * EMBED_DIM
    emb_col = channel_emb.reshape(flat, 1)
    out2d = pl.pallas_call(
        _bcast_kernel,
        grid=(B,),
        in_specs=[pl.BlockSpec((flat, 1), lambda i: (0, 0))],
        out_specs=pl.BlockSpec((flat, n_patches), lambda i: (i, 0)),
        out_shape=jax.ShapeDtypeStruct((B * flat, n_patches), channel_emb.dtype),
    )(emb_col)
    out_t = out2d.reshape(B, N_VARS, EMBED_DIM, n_patches)
    return out_t.transpose(0, 3, 1, 2)
